# speedup vs baseline: 2.6101x; 2.6101x over previous
import functools

import jax
import jax.numpy as jnp
from jax import lax
from jax.experimental import pallas as pl
from jax.experimental.pallas import tpu as pltpu
from jax.experimental.pallas import tpu_sc as plsc

B = 16
L = 2048
D = 256

NC = 2
NS = 16
NW = NC * NS
SEG = 2 * L + 1
HALF = L // 2
CHUNK = 128
NCHUNK = HALF // CHUNK


def _sc_interleave(item_values, action_values, contextual_values):
    mesh = plsc.VectorSubcoreMesh(core_axis_name="c", subcore_axis_name="s")
    out_rows = B * SEG

    @functools.partial(
        pl.kernel,
        mesh=mesh,
        out_type=jax.ShapeDtypeStruct((out_rows, D), jnp.float32),
        scratch_types=[
            pltpu.VMEM((CHUNK, D), jnp.float32),
            pltpu.VMEM((CHUNK, D), jnp.float32),
            pltpu.VMEM((CHUNK,), jnp.int32),
            pltpu.VMEM((CHUNK,), jnp.int32),
            pltpu.VMEM((1, D), jnp.float32),
            pltpu.SemaphoreType.DMA,
            pltpu.SemaphoreType.DMA,
        ],
    )
    def k(item_hbm, action_hbm, ctx_hbm, out_hbm,
          buf_i, buf_a, idx_i, idx_a, ctx_buf, sem_i, sem_a):
        wid = lax.axis_index("s") * NC + lax.axis_index("c")
        b = wid // 2
        h = wid % 2
        src_base = b * L + h * HALF
        out_base = b * SEG + 1 + h * (2 * HALF)
        lane = lax.iota(jnp.int32, 16)
        for c in range(NCHUNK):
            pltpu.sync_copy(item_hbm.at[pl.ds(src_base + c * CHUNK, CHUNK)],
                            buf_i)
            pltpu.sync_copy(action_hbm.at[pl.ds(src_base + c * CHUNK, CHUNK)],
                            buf_a)
            for i in range(CHUNK // 16):
                v = out_base + 2 * (c * CHUNK + i * 16) + 2 * lane
                idx_i[pl.ds(i * 16, 16)] = v
                idx_a[pl.ds(i * 16, 16)] = v + 1
            cp_i = pltpu.async_copy(buf_i, out_hbm.at[idx_i], sem_i)
            cp_a = pltpu.async_copy(buf_a, out_hbm.at[idx_a], sem_a)
            cp_i.wait()
            cp_a.wait()

        @pl.when(wid < B)
        def _():
            pltpu.sync_copy(ctx_hbm.at[pl.ds(wid, 1)], ctx_buf)
            pltpu.sync_copy(ctx_buf, out_hbm.at[pl.ds(wid * SEG, 1)])

    return k(item_values, action_values, contextual_values)


def kernel(item_values, action_values, contextual_values):
    out_values = _sc_interleave(item_values, action_values, contextual_values)
    out_lengths = jnp.full((B,), SEG, dtype=jnp.int32)
    out_offsets = (jnp.arange(B + 1, dtype=jnp.int32) * SEG).astype(jnp.int32)
    return out_values, out_lengths, out_offsets


# speedup vs baseline: 2.9817x; 1.1424x over previous
import functools

import jax
import jax.numpy as jnp
from jax import lax
from jax.experimental import pallas as pl
from jax.experimental.pallas import tpu as pltpu
from jax.experimental.pallas import tpu_sc as plsc

B = 16
L = 2048
D = 256

NC = 2
NS = 16
NW = NC * NS
SEG = 2 * L + 1
HALF = L // 2
CHUNK = 128
NCHUNK = HALF // CHUNK


def _sc_interleave(item_values, action_values, contextual_values):
    mesh = plsc.VectorSubcoreMesh(core_axis_name="c", subcore_axis_name="s")
    out_rows = B * SEG

    nbuf = 3
    nt = 2 * NCHUNK

    @functools.partial(
        pl.kernel,
        mesh=mesh,
        out_type=jax.ShapeDtypeStruct((out_rows, D), jnp.float32),
        scratch_types=(
            [pltpu.VMEM((CHUNK, D), jnp.float32) for _ in range(nbuf)]
            + [pltpu.VMEM((CHUNK,), jnp.int32) for _ in range(nbuf)]
            + [pltpu.VMEM((1, D), jnp.float32)]
            + [pltpu.SemaphoreType.DMA for _ in range(2 * nbuf)]
        ),
    )
    def k(item_hbm, action_hbm, ctx_hbm, out_hbm, *scr):
        bufs = scr[0:nbuf]
        idxs = scr[nbuf:2 * nbuf]
        ctx_buf = scr[2 * nbuf]
        gsem = scr[2 * nbuf + 1:2 * nbuf + 1 + nbuf]
        ssem = scr[2 * nbuf + 1 + nbuf:]
        wid = lax.axis_index("s") * NC + lax.axis_index("c")
        b = wid // 2
        h = wid % 2
        src_base = b * L + h * HALF
        out_base = b * SEG + 1 + h * (2 * HALF)
        lane = lax.iota(jnp.int32, 16)

        def start_gather(t):
            ref = item_hbm if t % 2 == 0 else action_hbm
            src = src_base + (t // 2) * CHUNK
            return pltpu.async_copy(ref.at[pl.ds(src, CHUNK)],
                                    bufs[t % nbuf], gsem[t % nbuf])

        gath = {t: start_gather(t) for t in range(min(nbuf, nt))}
        scat = {}
        for t in range(nt):
            s = t % nbuf
            if t >= 1 and t + 2 < nt:
                scat[t - 1].wait()
                gath[t + 2] = start_gather(t + 2)
            gath[t].wait()
            a = t % 2
            base = out_base + a + 2 * (t // 2) * CHUNK
            for i in range(CHUNK // 16):
                idxs[s][pl.ds(i * 16, 16)] = base + 2 * (i * 16 + lane)
            scat[t] = pltpu.async_copy(bufs[s], out_hbm.at[idxs[s]], ssem[s])
        for t in range(max(0, nt - nbuf), nt):
            scat[t].wait()

        @pl.when(wid < B)
        def _():
            pltpu.sync_copy(ctx_hbm.at[pl.ds(wid, 1)], ctx_buf)
            pltpu.sync_copy(ctx_buf, out_hbm.at[pl.ds(wid * SEG, 1)])

    return k(item_values, action_values, contextual_values)


def kernel(item_values, action_values, contextual_values):
    out_values = _sc_interleave(item_values, action_values, contextual_values)
    out_lengths = jnp.full((B,), SEG, dtype=jnp.int32)
    out_offsets = (jnp.arange(B + 1, dtype=jnp.int32) * SEG).astype(jnp.int32)
    return out_values, out_lengths, out_offsets
